# baseline (speedup 1.0000x reference)
import functools

import jax
import jax.numpy as jnp
from jax import lax
from jax.experimental import pallas as pl
from jax.experimental.pallas import tpu as pltpu
from jax.experimental.pallas import tpu_sc as plsc

_NC = 2
_NS = 16
_NW = _NC * _NS
_L = 16
_EPS = 1e-05


def _lane_sum(x):
    lanes = lax.iota(jnp.int32, _L)
    for sh in (8, 4, 2, 1):
        perm = lax.bitwise_xor(lanes, jnp.int32(sh))
        x = x + x.at[perm].get(mode="promise_in_bounds")
    return x


def _rsqrt_nr(v):
    i = plsc.bitcast(v, jnp.int32)
    i = jnp.int32(0x5F3759DF) - lax.shift_right_logical(i, jnp.int32(1))
    y = plsc.bitcast(i, jnp.float32)
    half = jnp.float32(0.5) * v
    for _ in range(3):
        y = y * (jnp.float32(1.5) - half * y * y)
    return y


def _make_sc_kernel(n, vocab, d, chunk):
    b_per_w = n // _NW
    n_chunks = b_per_w // chunk
    n_slices = d // _L
    mesh = plsc.VectorSubcoreMesh(
        core_axis_name="c", subcore_axis_name="s",
        num_cores=_NC, num_subcores=_NS)

    @functools.partial(
        pl.kernel,
        out_type=jax.ShapeDtypeStruct((n, d), jnp.float32),
        mesh=mesh,
        scratch_types=[
            pltpu.VMEM((b_per_w,), jnp.int32),
            pltpu.VMEM((d,), jnp.float32),
            pltpu.VMEM((chunk, d), jnp.float32),
            pltpu.SemaphoreType.DMA,
        ],
        compiler_params=pltpu.CompilerParams(needs_layout_passes=False),
    )
    def run(idx_hbm, tab_hbm, w_hbm, out_hbm, idx_v, w_v, buf, sem):
        wid = lax.axis_index("s") * _NC + lax.axis_index("c")
        base = wid * b_per_w
        pltpu.sync_copy(idx_hbm.at[pl.ds(base, b_per_w)], idx_v)
        pltpu.sync_copy(w_hbm, w_v)

        def chunk_body(c, carry):
            off = c * chunk
            pltpu.async_copy(
                tab_hbm.at[idx_v.at[pl.ds(off, chunk)]], buf, sem).wait()

            def row_body(r, carry2):
                acc = jnp.zeros((_L,), jnp.float32)
                for j in range(n_slices):
                    x = buf[r, pl.ds(j * _L, _L)]
                    acc = acc + x * x
                meanv = _lane_sum(acc) * jnp.float32(1.0 / d) + jnp.float32(_EPS)
                scale = _rsqrt_nr(meanv)
                for j in range(n_slices):
                    sl = pl.ds(j * _L, _L)
                    buf[r, sl] = buf[r, sl] * scale * w_v[sl]
                return carry2

            lax.fori_loop(0, chunk, row_body, 0, unroll=False)
            pltpu.sync_copy(buf, out_hbm.at[pl.ds(base + off, chunk)])
            return carry

        lax.fori_loop(0, n_chunks, chunk_body, 0, unroll=False)

    return run


def kernel(input_ids, table, rms_weight):
    batch, seq = input_ids.shape
    vocab, d = table.shape
    n = batch * seq
    idx = input_ids.reshape(n).astype(jnp.int32)
    sc = _make_sc_kernel(n, vocab, d, chunk=32)
    out = sc(idx, table, rms_weight.astype(jnp.float32))
    return out.reshape(batch, seq, d)


# speedup vs baseline: 1.2385x; 1.2385x over previous
import functools

import jax
import jax.numpy as jnp
from jax import lax
from jax.experimental import pallas as pl
from jax.experimental.pallas import tpu as pltpu
from jax.experimental.pallas import tpu_sc as plsc

_NC = 2
_NS = 16
_NW = _NC * _NS
_L = 16
_EPS = 1e-05


def _lane_sum(x):
    lanes = lax.iota(jnp.int32, _L)
    for sh in (8, 4, 2, 1):
        perm = lax.bitwise_xor(lanes, jnp.int32(sh))
        x = x + x.at[perm].get(mode="promise_in_bounds")
    return x


def _rsqrt_nr(v):
    i = plsc.bitcast(v, jnp.int32)
    i = jnp.int32(0x5F3759DF) - lax.shift_right_logical(i, jnp.int32(1))
    y = plsc.bitcast(i, jnp.float32)
    half = jnp.float32(0.5) * v
    for _ in range(3):
        y = y * (jnp.float32(1.5) - half * y * y)
    return y


def _make_sc_kernel(n, vocab, d, chunk):
    b_per_w = n // _NW
    n_chunks = b_per_w // chunk
    n_slices = d // _L
    assert b_per_w % chunk == 0 and n_chunks % 2 == 0
    mesh = plsc.VectorSubcoreMesh(
        core_axis_name="c", subcore_axis_name="s",
        num_cores=_NC, num_subcores=_NS)

    @functools.partial(
        pl.kernel,
        out_type=jax.ShapeDtypeStruct((n, d), jnp.float32),
        mesh=mesh,
        scratch_types=[
            pltpu.VMEM((b_per_w,), jnp.int32),
            pltpu.VMEM((d,), jnp.float32),
            pltpu.VMEM((chunk, d), jnp.float32),
            pltpu.VMEM((chunk, d), jnp.float32),
            pltpu.VMEM((chunk, d), jnp.float32),
            pltpu.VMEM((chunk, d), jnp.float32),
            pltpu.SemaphoreType.DMA,
            pltpu.SemaphoreType.DMA,
            pltpu.SemaphoreType.DMA,
            pltpu.SemaphoreType.DMA,
        ],
        compiler_params=pltpu.CompilerParams(needs_layout_passes=False),
    )
    def run(idx_hbm, tab_hbm, w_hbm, out_hbm,
            idx_v, w_v, g0, g1, s0, s1, gsem0, gsem1, ssem0, ssem1):
        wid = lax.axis_index("s") * _NC + lax.axis_index("c")
        base = wid * b_per_w
        pltpu.sync_copy(idx_hbm.at[pl.ds(base, b_per_w)], idx_v)
        pltpu.sync_copy(w_hbm, w_v)

        gbuf = (g0, g1)
        sbuf = (s0, s1)
        gsem = (gsem0, gsem1)
        ssem = (ssem0, ssem1)

        def start_gather(c, p):
            pltpu.async_copy(
                tab_hbm.at[idx_v.at[pl.ds(c * chunk, chunk)]],
                gbuf[p], gsem[p])

        def normalize(p):
            src, dst = gbuf[p], sbuf[p]

            def row_body(r, carry):
                acc = jnp.zeros((_L,), jnp.float32)
                for j in range(n_slices):
                    x = src[r, pl.ds(j * _L, _L)]
                    acc = acc + x * x
                meanv = _lane_sum(acc) * jnp.float32(1.0 / d) \
                    + jnp.float32(_EPS)
                scale = _rsqrt_nr(meanv)
                for j in range(n_slices):
                    sl = pl.ds(j * _L, _L)
                    dst[r, sl] = src[r, sl] * scale * w_v[sl]
                return carry

            lax.fori_loop(0, chunk, row_body, 0, unroll=False)

        start_gather(0, 0)
        start_gather(1, 1)

        def step(c, p):
            pltpu.make_async_copy(
                tab_hbm.at[idx_v.at[pl.ds(c * chunk, chunk)]],
                gbuf[p], gsem[p]).wait()
            normalize(p)

            @pl.when(c + 2 < n_chunks)
            def _():
                start_gather(c + 2, p)

            @pl.when(c >= 2)
            def _():
                pltpu.make_async_copy(
                    sbuf[p], out_hbm.at[pl.ds(base, chunk)], ssem[p]).wait()

            pltpu.async_copy(
                sbuf[p], out_hbm.at[pl.ds(base + c * chunk, chunk)], ssem[p])

        def pair_body(g, carry):
            step(2 * g, 0)
            step(2 * g + 1, 1)
            return carry

        lax.fori_loop(0, n_chunks // 2, pair_body, 0, unroll=False)

        pltpu.make_async_copy(
            s0, out_hbm.at[pl.ds(base, chunk)], ssem0).wait()
        pltpu.make_async_copy(
            s1, out_hbm.at[pl.ds(base, chunk)], ssem1).wait()

    return run


def kernel(input_ids, table, rms_weight):
    batch, seq = input_ids.shape
    vocab, d = table.shape
    n = batch * seq
    idx = input_ids.reshape(n).astype(jnp.int32)
    sc = _make_sc_kernel(n, vocab, d, chunk=16)
    out = sc(idx, table, rms_weight.astype(jnp.float32))
    return out.reshape(batch, seq, d)


# speedup vs baseline: 2.2068x; 1.7819x over previous
import functools

import jax
import jax.numpy as jnp
from jax import lax
from jax.experimental import pallas as pl
from jax.experimental.pallas import tpu as pltpu
from jax.experimental.pallas import tpu_sc as plsc

_NC = 2
_NS = 16
_NW = _NC * _NS
_EPS = 1e-05


_NSLOT = 4


def _make_sc_gather(n, d, chunk):
    b_per_w = n // _NW
    n_chunks = b_per_w // chunk
    assert b_per_w % chunk == 0 and n_chunks % _NSLOT == 0
    mesh = plsc.VectorSubcoreMesh(
        core_axis_name="c", subcore_axis_name="s",
        num_cores=_NC, num_subcores=_NS)

    @functools.partial(
        pl.kernel,
        out_type=jax.ShapeDtypeStruct((n, d), jnp.float32),
        mesh=mesh,
        scratch_types=[
            pltpu.VMEM((b_per_w,), jnp.int32),
        ]
        + [pltpu.VMEM((chunk, d), jnp.float32)] * _NSLOT
        + [pltpu.SemaphoreType.DMA] * (2 * _NSLOT),
        compiler_params=pltpu.CompilerParams(needs_layout_passes=False),
    )
    def run(idx_hbm, tab_hbm, out_hbm, idx_v, *bufsem):
        bufs = bufsem[:_NSLOT]
        gsem = bufsem[_NSLOT:2 * _NSLOT]
        ssem = bufsem[2 * _NSLOT:]
        wid = lax.axis_index("s") * _NC + lax.axis_index("c")
        base = wid * b_per_w
        pltpu.sync_copy(idx_hbm.at[pl.ds(base, b_per_w)], idx_v)

        def start_gather(c, p):
            pltpu.async_copy(
                tab_hbm.at[idx_v.at[pl.ds(c * chunk, chunk)]],
                bufs[p], gsem[p])

        def wait_gather(c, p):
            pltpu.make_async_copy(
                tab_hbm.at[idx_v.at[pl.ds(c * chunk, chunk)]],
                bufs[p], gsem[p]).wait()

        def start_store(c, p):
            pltpu.async_copy(
                bufs[p], out_hbm.at[pl.ds(base + c * chunk, chunk)], ssem[p])

        def wait_store(c, p):
            pltpu.make_async_copy(
                bufs[p], out_hbm.at[pl.ds(base + c * chunk, chunk)],
                ssem[p]).wait()

        for p in range(_NSLOT):
            start_gather(p, p)

        def step(c, p):
            wait_gather(c, p)
            start_store(c, p)

            @pl.when(c + _NSLOT < n_chunks)
            def _():
                wait_store(c, p)
                start_gather(c + _NSLOT, p)

        def ring_body(g, carry):
            for p in range(_NSLOT):
                step(g * _NSLOT + p, p)
            return carry

        lax.fori_loop(0, n_chunks // _NSLOT, ring_body, 0, unroll=False)

        for p in range(_NSLOT):
            wait_store(n_chunks - _NSLOT + p, p)

    return run


def _norm_body(w_ref, x_ref, o_ref, *, d):
    x = x_ref[...]
    ms = jnp.mean(x * x, axis=-1, keepdims=True)
    o_ref[...] = x * lax.rsqrt(ms + _EPS) * w_ref[...]


def _make_tc_norm(n, d, block_rows):
    assert n % block_rows == 0
    grid = (n // block_rows,)
    return pl.pallas_call(
        functools.partial(_norm_body, d=d),
        grid=grid,
        in_specs=[
            pl.BlockSpec((1, d), lambda i: (0, 0)),
            pl.BlockSpec((block_rows, d), lambda i: (i, 0)),
        ],
        out_specs=pl.BlockSpec((block_rows, d), lambda i: (i, 0)),
        out_shape=jax.ShapeDtypeStruct((n, d), jnp.float32),
    )


def kernel(input_ids, table, rms_weight):
    batch, seq = input_ids.shape
    vocab, d = table.shape
    n = batch * seq
    idx = input_ids.reshape(n).astype(jnp.int32)
    gathered = _make_sc_gather(n, d, chunk=16)(idx, table)
    gathered = lax.optimization_barrier(gathered)
    out = _make_tc_norm(n, d, block_rows=256)(
        rms_weight.astype(jnp.float32).reshape(1, d), gathered)
    return out.reshape(batch, seq, d)


# speedup vs baseline: 2.7746x; 1.2573x over previous
import functools

import jax
import jax.numpy as jnp
from jax import lax
from jax.experimental import pallas as pl
from jax.experimental.pallas import tpu as pltpu
from jax.experimental.pallas import tpu_sc as plsc

_NC = 2
_NS = 16
_NW = _NC * _NS
_EPS = 1e-05


_NSLOT = 4


def _make_sc_gather(n, d, chunk):
    b_per_w = n // _NW
    n_chunks = b_per_w // chunk
    assert b_per_w % chunk == 0 and n_chunks % _NSLOT == 0
    mesh = plsc.VectorSubcoreMesh(
        core_axis_name="c", subcore_axis_name="s",
        num_cores=_NC, num_subcores=_NS)

    @functools.partial(
        pl.kernel,
        out_type=jax.ShapeDtypeStruct((n, d), jnp.float32),
        mesh=mesh,
        scratch_types=[
            pltpu.VMEM((b_per_w,), jnp.int32),
        ]
        + [pltpu.VMEM((chunk, d), jnp.float32)] * _NSLOT
        + [pltpu.SemaphoreType.DMA] * (2 * _NSLOT),
        compiler_params=pltpu.CompilerParams(needs_layout_passes=False),
    )
    def run(idx_hbm, tab_hbm, out_hbm, idx_v, *bufsem):
        bufs = bufsem[:_NSLOT]
        gsem = bufsem[_NSLOT:2 * _NSLOT]
        ssem = bufsem[2 * _NSLOT:]
        wid = lax.axis_index("s") * _NC + lax.axis_index("c")
        base = wid * b_per_w
        pltpu.sync_copy(idx_hbm.at[pl.ds(base, b_per_w)], idx_v)

        def start_gather(c, p):
            pltpu.async_copy(
                tab_hbm.at[idx_v.at[pl.ds(c * chunk, chunk)]],
                bufs[p], gsem[p])

        def wait_gather(c, p):
            pltpu.make_async_copy(
                tab_hbm.at[idx_v.at[pl.ds(c * chunk, chunk)]],
                bufs[p], gsem[p]).wait()

        def start_store(c, p):
            pltpu.async_copy(
                bufs[p], out_hbm.at[pl.ds(base + c * chunk, chunk)], ssem[p])

        def wait_store(c, p):
            pltpu.make_async_copy(
                bufs[p], out_hbm.at[pl.ds(base + c * chunk, chunk)],
                ssem[p]).wait()

        for p in range(_NSLOT):
            start_gather(p, p)

        def step(c, p):
            wait_gather(c, p)
            start_store(c, p)

            @pl.when(c + _NSLOT < n_chunks)
            def _():
                wait_store(c, p)
                start_gather(c + _NSLOT, p)

        def ring_body(g, carry):
            for p in range(_NSLOT):
                step(g * _NSLOT + p, p)
            return carry

        lax.fori_loop(0, n_chunks // _NSLOT, ring_body, 0, unroll=False)

        for p in range(_NSLOT):
            wait_store(n_chunks - _NSLOT + p, p)

    return run


def _norm_body(w_ref, x_ref, o_ref, *, d):
    x = x_ref[...]
    ms = jnp.mean(x * x, axis=-1, keepdims=True)
    o_ref[...] = x * lax.rsqrt(ms + _EPS) * w_ref[...]


def _make_tc_norm(n, d, block_rows):
    assert n % block_rows == 0
    grid = (n // block_rows,)
    return pl.pallas_call(
        functools.partial(_norm_body, d=d),
        grid=grid,
        in_specs=[
            pl.BlockSpec((1, d), lambda i: (0, 0)),
            pl.BlockSpec((block_rows, d), lambda i: (i, 0)),
        ],
        out_specs=pl.BlockSpec((block_rows, d), lambda i: (i, 0)),
        out_shape=jax.ShapeDtypeStruct((n, d), jnp.float32),
    )


def kernel(input_ids, table, rms_weight):
    batch, seq = input_ids.shape
    vocab, d = table.shape
    n = batch * seq
    idx = input_ids.reshape(n).astype(jnp.int32)
    gathered = _make_sc_gather(n, d, chunk=16)(idx, table)
    gathered = lax.optimization_barrier(gathered)
    out = _make_tc_norm(n, d, block_rows=1024)(
        rms_weight.astype(jnp.float32).reshape(1, d), gathered)
    return out.reshape(batch, seq, d)
